# baseline (speedup 1.0000x reference)
import jax
import jax.numpy as jnp
from jax import lax
from jax.experimental import pallas as pl
from jax.experimental.pallas import tpu as pltpu
from jax.experimental.pallas import tpu_sc as plsc

_B, _L, _OUT = 128, 1425, 256
_N = _B * _L

_NC, _NS = 2, 16
_NW = _NC * _NS
_BPW = _B // _NW
_CHUNK = 128
_FULL = _L // _CHUNK
_TAIL = _L - _FULL * _CHUNK
_IDXB = 1440
_NBUF = 3
_NFK = _BPW * _FULL

_MB = 4
_STEPS = _B // _MB


def _sc_gather_body(table_hbm, idx_hbm, out_hbm, idx_v,
                    bufs, tbuf, sems, tsem):
    wid = lax.axis_index("s") * _NC + lax.axis_index("c")
    b0 = wid * _BPW

    pltpu.sync_copy(idx_hbm.at[pl.ds(wid * _BPW * _IDXB, _BPW * _IDXB)],
                    idx_v)

    def start_g(k, b):
        bl = k // _FULL
        c = k % _FULL
        iref = idx_v.at[pl.ds(bl * _IDXB + c * _CHUNK, _CHUNK)]
        pltpu.async_copy(table_hbm.at[iref], bufs[b], sems[b])

    def wait_g(b):
        iref = idx_v.at[pl.ds(0, _CHUNK)]
        pltpu.make_async_copy(table_hbm.at[iref], bufs[b], sems[b]).wait()

    for b in range(_NBUF):
        start_g(b, b)

    def body(k, carry):
        bl = k // _FULL
        c = k % _FULL
        for b in range(_NBUF):
            @pl.when(k % _NBUF == b)
            def _():
                wait_g(b)
                pltpu.sync_copy(
                    bufs[b], out_hbm.at[b0 + bl, pl.ds(c * _CHUNK, _CHUNK)])

                @pl.when(k + _NBUF < _NFK)
                def _():
                    start_g(k + _NBUF, b)
        return carry

    lax.fori_loop(0, _NFK, body, 0)

    for bl in range(_BPW):
        iref = idx_v.at[pl.ds(bl * _IDXB + _FULL * _CHUNK, _TAIL)]
        pltpu.async_copy(table_hbm.at[iref], tbuf, tsem).wait()
        pltpu.sync_copy(
            tbuf, out_hbm.at[b0 + bl, pl.ds(_FULL * _CHUNK, _TAIL)])


def _sc_gather(table, idx_flat):
    mesh = plsc.VectorSubcoreMesh(core_axis_name="c", subcore_axis_name="s")
    fn = pl.kernel(
        lambda table_hbm, idx_hbm, out_hbm, idx_v, u0, u1, u2, tb, s0, s1,
        s2, ts: _sc_gather_body(table_hbm, idx_hbm, out_hbm, idx_v,
                                (u0, u1, u2), tb, (s0, s1, s2), ts),
        out_type=jax.ShapeDtypeStruct((_B, _L, 128), jnp.float32),
        mesh=mesh,
        scratch_types=[
            pltpu.VMEM((_BPW * _IDXB,), jnp.int32),
            pltpu.VMEM((_CHUNK, 128), jnp.float32),
            pltpu.VMEM((_CHUNK, 128), jnp.float32),
            pltpu.VMEM((_CHUNK, 128), jnp.float32),
            pltpu.VMEM((_TAIL, 128), jnp.float32),
            pltpu.SemaphoreType.DMA,
            pltpu.SemaphoreType.DMA,
            pltpu.SemaphoreType.DMA,
            pltpu.SemaphoreType.DMA,
        ],
    )
    return fn(table, idx_flat)


def _extract_body(x_ref, o_ref):
    x = x_ref[...]
    rows = [x[bi, :, 0] for bi in range(8)]
    stacked = jnp.stack(rows, axis=0)
    padded = jnp.concatenate(
        [stacked, jnp.zeros((8, _IDXB - _L), jnp.float32)], axis=1)
    o_ref[...] = padded.astype(jnp.int32)[None]


def _extract_idx(x_omic):
    out = pl.pallas_call(
        _extract_body,
        grid=(_B // 8,),
        in_specs=[pl.BlockSpec((8, _L, 9), lambda i: (i, 0, 0))],
        out_specs=pl.BlockSpec((1, 8, _IDXB), lambda i: (i, 0, 0)),
        out_shape=jax.ShapeDtypeStruct((_B // 8, 8, _IDXB), jnp.int32),
    )(x_omic)
    return out.reshape(_B * _IDXB)


def _tc_body(x_ref, hv_ref, evc_ref, efn_ref, w_ref, b_ref, o_ref, wf_ref):
    @pl.when(pl.program_id(0) == 0)
    def _():
        wvc = jnp.dot(evc_ref[...], w_ref[128:160, :],
                      preferred_element_type=jnp.float32)
        wfn = jnp.dot(efn_ref[...], w_ref[160:192, :],
                      preferred_element_type=jnp.float32)
        z7 = jnp.zeros((7, 256), jnp.float32)
        z15 = jnp.zeros((15, 256), jnp.float32)
        wf_ref[...] = jnp.concatenate(
            [w_ref[0:128, :], wvc, z7, wfn, z7, w_ref[192:193, :], z15], axis=0)

    x = x_ref[...]
    hv = hv_ref[...]
    iota = lax.broadcasted_iota(jnp.int32, (_MB, _L, 128), 2).astype(jnp.float32)
    a = (x[..., 1:2] == iota).astype(jnp.float32)
    sixth = jnp.float32(1.0 / 6.0)
    for k in range(6):
        a = a + jnp.where(x[..., 2 + k:3 + k] == iota - 40.0, sixth, 0.0)
    a = a + jnp.where(iota == 112.0, x[..., 8:9], 0.0)
    h2 = jnp.concatenate([hv, a], axis=2)
    y = lax.dot_general(h2, wf_ref[...],
                        (((2,), (0,)), ((), ())),
                        preferred_element_type=jnp.float32) + b_ref[...]
    o_ref[...] = jnp.where(y > 0.0, y, jnp.exp(jnp.minimum(y, 0.0)) - 1.0)


def _tc_call(x_omic, hvar, emb_vc, emb_func, w, b):
    return pl.pallas_call(
        _tc_body,
        grid=(_STEPS,),
        in_specs=[
            pl.BlockSpec((_MB, _L, 9), lambda i: (i, 0, 0)),
            pl.BlockSpec((_MB, _L, 128), lambda i: (i, 0, 0)),
            pl.BlockSpec((33, 32), lambda i: (0, 0)),
            pl.BlockSpec((65, 32), lambda i: (0, 0)),
            pl.BlockSpec((193, 256), lambda i: (0, 0)),
            pl.BlockSpec((1, 256), lambda i: (0, 0)),
        ],
        out_specs=pl.BlockSpec((_MB, _L, _OUT), lambda i: (i, 0, 0)),
        out_shape=jax.ShapeDtypeStruct((_B, _L, _OUT), jnp.float32),
        scratch_shapes=[pltpu.VMEM((256, 256), jnp.float32)],
    )(x_omic, hvar, emb_vc, emb_func, w, b)


def kernel(x_omic, emb_var, emb_vc, emb_func, W, b):
    idx = _extract_idx(x_omic)
    hvar = _sc_gather(emb_var, idx)
    return _tc_call(x_omic, hvar, emb_vc, emb_func, W, b.reshape(1, _OUT))


# speedup vs baseline: 1.4649x; 1.4649x over previous
import jax
import jax.numpy as jnp
from jax import lax
from jax.experimental import pallas as pl
from jax.experimental.pallas import tpu as pltpu
from jax.experimental.pallas import tpu_sc as plsc

_B, _L, _OUT = 128, 1425, 256
_N = _B * _L

_NC, _NS = 2, 16
_NW = _NC * _NS
_BPW = _B // _NW
_CHUNK = 128
_FULL = _L // _CHUNK
_TAIL = _L - _FULL * _CHUNK
_IDXB = 1440
_NBUF = 3
_NFK = _BPW * _FULL

_MB = 1
_STEPS = _B // _MB


def _sc_gather_body(table_hbm, idx_hbm, out_hbm, idx_v,
                    bufs, tbuf, sems, tsem):
    wid = lax.axis_index("s") * _NC + lax.axis_index("c")
    b0 = wid * _BPW

    pltpu.sync_copy(idx_hbm.at[pl.ds(wid * _BPW * _IDXB, _BPW * _IDXB)],
                    idx_v)

    def start_g(k, b):
        bl = k // _FULL
        c = k % _FULL
        iref = idx_v.at[pl.ds(bl * _IDXB + c * _CHUNK, _CHUNK)]
        pltpu.async_copy(table_hbm.at[iref], bufs[b], sems[b])

    def wait_g(b):
        iref = idx_v.at[pl.ds(0, _CHUNK)]
        pltpu.make_async_copy(table_hbm.at[iref], bufs[b], sems[b]).wait()

    for b in range(_NBUF):
        start_g(b, b)

    def body(k, carry):
        bl = k // _FULL
        c = k % _FULL
        for b in range(_NBUF):
            @pl.when(k % _NBUF == b)
            def _():
                wait_g(b)
                pltpu.sync_copy(
                    bufs[b], out_hbm.at[b0 + bl, pl.ds(c * _CHUNK, _CHUNK)])

                @pl.when(k + _NBUF < _NFK)
                def _():
                    start_g(k + _NBUF, b)
        return carry

    lax.fori_loop(0, _NFK, body, 0)

    for bl in range(_BPW):
        iref = idx_v.at[pl.ds(bl * _IDXB + _FULL * _CHUNK, _TAIL)]
        pltpu.async_copy(table_hbm.at[iref], tbuf, tsem).wait()
        pltpu.sync_copy(
            tbuf, out_hbm.at[b0 + bl, pl.ds(_FULL * _CHUNK, _TAIL)])


def _sc_gather(table, idx_flat):
    mesh = plsc.VectorSubcoreMesh(core_axis_name="c", subcore_axis_name="s")
    fn = pl.kernel(
        lambda table_hbm, idx_hbm, out_hbm, idx_v, u0, u1, u2, tb, s0, s1,
        s2, ts: _sc_gather_body(table_hbm, idx_hbm, out_hbm, idx_v,
                                (u0, u1, u2), tb, (s0, s1, s2), ts),
        out_type=jax.ShapeDtypeStruct((_B, _L, 128), jnp.float32),
        mesh=mesh,
        scratch_types=[
            pltpu.VMEM((_BPW * _IDXB,), jnp.int32),
            pltpu.VMEM((_CHUNK, 128), jnp.float32),
            pltpu.VMEM((_CHUNK, 128), jnp.float32),
            pltpu.VMEM((_CHUNK, 128), jnp.float32),
            pltpu.VMEM((_TAIL, 128), jnp.float32),
            pltpu.SemaphoreType.DMA,
            pltpu.SemaphoreType.DMA,
            pltpu.SemaphoreType.DMA,
            pltpu.SemaphoreType.DMA,
        ],
    )
    return fn(table, idx_flat)


def _extract_body(x_ref, o_ref):
    x = x_ref[...]
    rows = [x[bi, :, 0] for bi in range(8)]
    stacked = jnp.stack(rows, axis=0)
    padded = jnp.concatenate(
        [stacked, jnp.zeros((8, _IDXB - _L), jnp.float32)], axis=1)
    o_ref[...] = padded.astype(jnp.int32)[None]


def _extract_idx(x_omic):
    out = pl.pallas_call(
        _extract_body,
        grid=(_B // 8,),
        in_specs=[pl.BlockSpec((8, _L, 9), lambda i: (i, 0, 0))],
        out_specs=pl.BlockSpec((1, 8, _IDXB), lambda i: (i, 0, 0)),
        out_shape=jax.ShapeDtypeStruct((_B // 8, 8, _IDXB), jnp.int32),
    )(x_omic)
    return out.reshape(_B * _IDXB)


def _tc_body(x_ref, hv_ref, evc_ref, efn_ref, w_ref, b_ref, o_ref, wf_ref):
    @pl.when(pl.program_id(0) == 0)
    def _():
        wvc = jnp.dot(evc_ref[...], w_ref[128:160, :],
                      preferred_element_type=jnp.float32)
        wfn = jnp.dot(efn_ref[...], w_ref[160:192, :],
                      preferred_element_type=jnp.float32)
        z7 = jnp.zeros((7, 256), jnp.float32)
        z15 = jnp.zeros((15, 256), jnp.float32)
        wf_ref[...] = jnp.concatenate(
            [w_ref[0:128, :], wvc, z7, wfn, z7, w_ref[192:193, :], z15], axis=0)

    x = x_ref[0]
    hv = hv_ref[0]
    iota = lax.broadcasted_iota(jnp.int32, (_L, 128), 1).astype(jnp.float32)
    a = (x[:, 1:2] == iota).astype(jnp.float32)
    sixth = jnp.float32(1.0 / 6.0)
    for k in range(6):
        a = a + jnp.where(x[:, 2 + k:3 + k] == iota - 40.0, sixth, 0.0)
    a = a + jnp.where(iota == 112.0, x[:, 8:9], 0.0)
    y = (jnp.dot(hv, wf_ref[0:128, :], preferred_element_type=jnp.float32)
         + jnp.dot(a, wf_ref[128:256, :], preferred_element_type=jnp.float32)
         + b_ref[...])
    o_ref[0] = jnp.where(y > 0.0, y, jnp.exp(jnp.minimum(y, 0.0)) - 1.0)


def _tc_call(x_omic, hvar, emb_vc, emb_func, w, b):
    return pl.pallas_call(
        _tc_body,
        grid=(_STEPS,),
        in_specs=[
            pl.BlockSpec((_MB, _L, 9), lambda i: (i, 0, 0)),
            pl.BlockSpec((_MB, _L, 128), lambda i: (i, 0, 0)),
            pl.BlockSpec((33, 32), lambda i: (0, 0)),
            pl.BlockSpec((65, 32), lambda i: (0, 0)),
            pl.BlockSpec((193, 256), lambda i: (0, 0)),
            pl.BlockSpec((1, 256), lambda i: (0, 0)),
        ],
        out_specs=pl.BlockSpec((_MB, _L, _OUT), lambda i: (i, 0, 0)),
        out_shape=jax.ShapeDtypeStruct((_B, _L, _OUT), jnp.float32),
        scratch_shapes=[pltpu.VMEM((256, 256), jnp.float32)],
    )(x_omic, hvar, emb_vc, emb_func, w, b)


def kernel(x_omic, emb_var, emb_vc, emb_func, W, b):
    idx = _extract_idx(x_omic)
    hvar = _sc_gather(emb_var, idx)
    return _tc_call(x_omic, hvar, emb_vc, emb_func, W, b.reshape(1, _OUT))


# speedup vs baseline: 1.6505x; 1.1267x over previous
import jax
import jax.numpy as jnp
from jax import lax
from jax.experimental import pallas as pl
from jax.experimental.pallas import tpu as pltpu
from jax.experimental.pallas import tpu_sc as plsc

_B, _L, _OUT = 128, 1425, 256
_N = _B * _L

_NC, _NS = 2, 16
_NW = _NC * _NS
_BPW = _B // _NW
_CHUNK = 128
_FULL = _L // _CHUNK
_TAIL = _L - _FULL * _CHUNK
_IDXB = 1440
_NBUF = 3
_NFK = _BPW * _FULL

_MB = 8
_STEPS = _B // _MB


def _sc_gather_body(table_hbm, idx_hbm, out_hbm, idx_v,
                    bufs, tbuf, sems, tsem):
    wid = lax.axis_index("s") * _NC + lax.axis_index("c")
    b0 = wid * _BPW

    pltpu.sync_copy(idx_hbm.at[pl.ds(wid * _BPW * _IDXB, _BPW * _IDXB)],
                    idx_v)

    def start_g(k, b):
        bl = k // _FULL
        c = k % _FULL
        iref = idx_v.at[pl.ds(bl * _IDXB + c * _CHUNK, _CHUNK)]
        pltpu.async_copy(table_hbm.at[iref], bufs[b], sems[b])

    def wait_g(b):
        iref = idx_v.at[pl.ds(0, _CHUNK)]
        pltpu.make_async_copy(table_hbm.at[iref], bufs[b], sems[b]).wait()

    for b in range(_NBUF):
        start_g(b, b)

    def body(k, carry):
        bl = k // _FULL
        c = k % _FULL
        for b in range(_NBUF):
            @pl.when(k % _NBUF == b)
            def _():
                wait_g(b)
                pltpu.sync_copy(
                    bufs[b], out_hbm.at[b0 + bl, pl.ds(c * _CHUNK, _CHUNK)])

                @pl.when(k + _NBUF < _NFK)
                def _():
                    start_g(k + _NBUF, b)
        return carry

    lax.fori_loop(0, _NFK, body, 0)

    for bl in range(_BPW):
        iref = idx_v.at[pl.ds(bl * _IDXB + _FULL * _CHUNK, _TAIL)]
        pltpu.async_copy(table_hbm.at[iref], tbuf, tsem).wait()
        pltpu.sync_copy(
            tbuf, out_hbm.at[b0 + bl, pl.ds(_FULL * _CHUNK, _TAIL)])


def _sc_gather(table, idx_flat):
    mesh = plsc.VectorSubcoreMesh(core_axis_name="c", subcore_axis_name="s")
    fn = pl.kernel(
        lambda table_hbm, idx_hbm, out_hbm, idx_v, u0, u1, u2, tb, s0, s1,
        s2, ts: _sc_gather_body(table_hbm, idx_hbm, out_hbm, idx_v,
                                (u0, u1, u2), tb, (s0, s1, s2), ts),
        out_type=jax.ShapeDtypeStruct((_B, _L, 128), jnp.float32),
        mesh=mesh,
        scratch_types=[
            pltpu.VMEM((_BPW * _IDXB,), jnp.int32),
            pltpu.VMEM((_CHUNK, 128), jnp.float32),
            pltpu.VMEM((_CHUNK, 128), jnp.float32),
            pltpu.VMEM((_CHUNK, 128), jnp.float32),
            pltpu.VMEM((_TAIL, 128), jnp.float32),
            pltpu.SemaphoreType.DMA,
            pltpu.SemaphoreType.DMA,
            pltpu.SemaphoreType.DMA,
            pltpu.SemaphoreType.DMA,
        ],
    )
    return fn(table, idx_flat)


def _extract_idx_body(x_ref, oi_ref):
    rows = [x_ref[bi, :, 0] for bi in range(8)]
    stacked = jnp.stack(rows, axis=0)
    padded = jnp.concatenate(
        [stacked, jnp.zeros((8, _IDXB - _L), jnp.float32)], axis=1)
    oi_ref[...] = padded.astype(jnp.int32)[None]


def _extract_idx(x_omic):
    return pl.pallas_call(
        _extract_idx_body,
        grid=(_B // 8,),
        in_specs=[pl.BlockSpec((8, _L, 9), lambda i: (i, 0, 0))],
        out_specs=pl.BlockSpec((1, 8, _IDXB), lambda i: (i, 0, 0)),
        out_shape=jax.ShapeDtypeStruct((_B // 8, 8, _IDXB), jnp.int32),
    )(x_omic)


def _extract_a_body(x_ref, oa_ref):
    iota = lax.broadcasted_iota(jnp.int32, (1, 128), 1).astype(jnp.float32)
    a = (x_ref[0, :, 1:2] == iota).astype(jnp.float32)
    for k in range(6):
        a = a + jnp.where(x_ref[0, :, 2 + k:3 + k] == iota - 40.0, 1.0, 0.0)
    a = a + jnp.where(iota == 112.0, x_ref[0, :, 8:9], 0.0)
    oa_ref[0] = a.astype(jnp.bfloat16)


def _extract_a(x_omic):
    return pl.pallas_call(
        _extract_a_body,
        grid=(_B,),
        in_specs=[pl.BlockSpec((1, _L, 9), lambda i: (i, 0, 0))],
        out_specs=pl.BlockSpec((1, _L, 128), lambda i: (i, 0, 0)),
        out_shape=jax.ShapeDtypeStruct((_B, _L, 128), jnp.bfloat16),
    )(x_omic)


def _tc_body(hv_ref, a_ref, evc_ref, efn_ref, w_ref, b_ref, o_ref,
             wf1_ref, wf2_ref):
    @pl.when((pl.program_id(0) == 0) & (pl.program_id(1) == 0))
    def _():
        wvc = jnp.dot(evc_ref[...], w_ref[128:160, :],
                      preferred_element_type=jnp.float32)
        wfn = jnp.dot(efn_ref[...], w_ref[160:192, :],
                      preferred_element_type=jnp.float32) * (1.0 / 6.0)
        z7 = jnp.zeros((7, 256), jnp.float32)
        z15 = jnp.zeros((15, 256), jnp.float32)
        lower = jnp.concatenate(
            [wvc, z7, wfn, z7, w_ref[192:193, :], z15], axis=0)
        for h in range(2):
            wf1_ref[h] = w_ref[0:128, pl.ds(h * 128, 128)]
            wf2_ref[h] = lower[:, h * 128:(h + 1) * 128].astype(jnp.bfloat16)

    h = pl.program_id(1)
    for bi in range(_MB):
        hv = hv_ref[bi]
        a = a_ref[bi]
        y = (jnp.dot(hv, wf1_ref[h], preferred_element_type=jnp.float32)
             + jnp.dot(a, wf2_ref[h], preferred_element_type=jnp.float32)
             + b_ref[...])
        o_ref[:, bi, :] = jnp.where(y > 0.0, y,
                                    jnp.exp(jnp.minimum(y, 0.0)) - 1.0)


def _tc_call(hvar, ab, emb_vc, emb_func, w, b):
    return pl.pallas_call(
        _tc_body,
        grid=(_STEPS, 2),
        in_specs=[
            pl.BlockSpec((_MB, _L, 128), lambda i, h: (i, 0, 0)),
            pl.BlockSpec((_MB, _L, 128), lambda i, h: (i, 0, 0)),
            pl.BlockSpec((33, 32), lambda i, h: (0, 0)),
            pl.BlockSpec((65, 32), lambda i, h: (0, 0)),
            pl.BlockSpec((193, 256), lambda i, h: (0, 0)),
            pl.BlockSpec((1, 128), lambda i, h: (0, h)),
        ],
        out_specs=pl.BlockSpec((_L, _MB, 128), lambda i, h: (0, i, h)),
        out_shape=jax.ShapeDtypeStruct((_L, _B, _OUT), jnp.float32),
        scratch_shapes=[
            pltpu.VMEM((2, 128, 128), jnp.float32),
            pltpu.VMEM((2, 128, 128), jnp.bfloat16),
        ],
    )(hvar, ab, emb_vc, emb_func, w, b)


def kernel(x_omic, emb_var, emb_vc, emb_func, W, b):
    idx3 = _extract_idx(x_omic)
    ab = _extract_a(x_omic)
    idx = idx3.reshape(_B * _IDXB)
    hvar = _sc_gather(emb_var, idx)
    out = _tc_call(hvar, ab, emb_vc, emb_func, W, b.reshape(1, _OUT))
    return jnp.transpose(out, (1, 0, 2))


# speedup vs baseline: 1.8549x; 1.1239x over previous
import jax
import jax.numpy as jnp
from jax import lax
from jax.experimental import pallas as pl
from jax.experimental.pallas import tpu as pltpu
from jax.experimental.pallas import tpu_sc as plsc

_B, _L, _OUT = 128, 1425, 256
_N = _B * _L

_NC, _NS = 2, 16
_NW = _NC * _NS
_BPW = _B // _NW
_CHUNK = 128
_FULL = _L // _CHUNK
_TAIL = _L - _FULL * _CHUNK
_IDXB = 1440
_NBUF = 3
_NFK = _BPW * _FULL

_MB = 8
_STEPS = _B // _MB


def _sc_gather_body(table_hbm, idx_hbm, out_hbm, idx_v,
                    bufs, tbuf, sems, tsem):
    wid = lax.axis_index("s") * _NC + lax.axis_index("c")
    b0 = wid * _BPW

    pltpu.sync_copy(idx_hbm.at[pl.ds(wid * _BPW * _IDXB, _BPW * _IDXB)],
                    idx_v)

    def start_g(k, b):
        bl = k // _FULL
        c = k % _FULL
        iref = idx_v.at[pl.ds(bl * _IDXB + c * _CHUNK, _CHUNK)]
        pltpu.async_copy(table_hbm.at[iref], bufs[b], sems[b])

    def wait_g(b):
        iref = idx_v.at[pl.ds(0, _CHUNK)]
        pltpu.make_async_copy(table_hbm.at[iref], bufs[b], sems[b]).wait()

    for b in range(_NBUF):
        start_g(b, b)

    def body(k, carry):
        bl = k // _FULL
        c = k % _FULL
        for b in range(_NBUF):
            @pl.when(k % _NBUF == b)
            def _():
                wait_g(b)
                pltpu.sync_copy(
                    bufs[b], out_hbm.at[b0 + bl, pl.ds(c * _CHUNK, _CHUNK)])

                @pl.when(k + _NBUF < _NFK)
                def _():
                    start_g(k + _NBUF, b)
        return carry

    lax.fori_loop(0, _NFK, body, 0)

    for bl in range(_BPW):
        iref = idx_v.at[pl.ds(bl * _IDXB + _FULL * _CHUNK, _TAIL)]
        pltpu.async_copy(table_hbm.at[iref], tbuf, tsem).wait()
        pltpu.sync_copy(
            tbuf, out_hbm.at[b0 + bl, pl.ds(_FULL * _CHUNK, _TAIL)])


def _sc_gather(table, idx_flat):
    mesh = plsc.VectorSubcoreMesh(core_axis_name="c", subcore_axis_name="s")
    fn = pl.kernel(
        lambda table_hbm, idx_hbm, out_hbm, idx_v, u0, u1, u2, tb, s0, s1,
        s2, ts: _sc_gather_body(table_hbm, idx_hbm, out_hbm, idx_v,
                                (u0, u1, u2), tb, (s0, s1, s2), ts),
        out_type=jax.ShapeDtypeStruct((_B, _L, 128), jnp.float32),
        mesh=mesh,
        scratch_types=[
            pltpu.VMEM((_BPW * _IDXB,), jnp.int32),
            pltpu.VMEM((_CHUNK, 128), jnp.float32),
            pltpu.VMEM((_CHUNK, 128), jnp.float32),
            pltpu.VMEM((_CHUNK, 128), jnp.float32),
            pltpu.VMEM((_TAIL, 128), jnp.float32),
            pltpu.SemaphoreType.DMA,
            pltpu.SemaphoreType.DMA,
            pltpu.SemaphoreType.DMA,
            pltpu.SemaphoreType.DMA,
        ],
    )
    return fn(table, idx_flat)


def _extract_idx_body(x_ref, oi_ref):
    rows = [x_ref[bi, :, 0] for bi in range(8)]
    stacked = jnp.stack(rows, axis=0)
    padded = jnp.concatenate(
        [stacked, jnp.zeros((8, _IDXB - _L), jnp.float32)], axis=1)
    oi_ref[...] = padded.astype(jnp.int32)[None]


def _extract_idx(x_omic):
    return pl.pallas_call(
        _extract_idx_body,
        grid=(_B // 8,),
        in_specs=[pl.BlockSpec((8, _L, 9), lambda i: (i, 0, 0))],
        out_specs=pl.BlockSpec((1, 8, _IDXB), lambda i: (i, 0, 0)),
        out_shape=jax.ShapeDtypeStruct((_B // 8, 8, _IDXB), jnp.int32),
    )(x_omic)


def _extract_a_body(x_ref, oa_ref):
    iota = lax.broadcasted_iota(
        jnp.int32, (1, 128), 1).astype(jnp.bfloat16)
    one = jnp.bfloat16(1.0)
    zero = jnp.bfloat16(0.0)
    xb = [x_ref[0, :, k:k + 1].astype(jnp.bfloat16) for k in range(1, 9)]
    tgt = iota - jnp.bfloat16(40.0)
    t = [jnp.where(xb[0] == iota, one, zero)]
    t += [jnp.where(xb[1 + k] == tgt, one, zero) for k in range(6)]
    t += [xb[7] * jnp.where(iota == jnp.bfloat16(112.0), one, zero)]
    oa_ref[0] = ((t[0] + t[1]) + (t[2] + t[3])) + ((t[4] + t[5])
                                                   + (t[6] + t[7]))


def _extract_a(x_omic):
    return pl.pallas_call(
        _extract_a_body,
        grid=(_B,),
        in_specs=[pl.BlockSpec((1, _L, 9), lambda i: (i, 0, 0))],
        out_specs=pl.BlockSpec((1, _L, 128), lambda i: (i, 0, 0)),
        out_shape=jax.ShapeDtypeStruct((_B, _L, 128), jnp.bfloat16),
    )(x_omic)


def _tc_body(hv_ref, a_ref, evc_ref, efn_ref, w_ref, b_ref, o_ref,
             wf1_ref, wf2_ref):
    @pl.when((pl.program_id(0) == 0) & (pl.program_id(1) == 0))
    def _():
        wvc = jnp.dot(evc_ref[...], w_ref[128:160, :],
                      preferred_element_type=jnp.float32)
        wfn = jnp.dot(efn_ref[...], w_ref[160:192, :],
                      preferred_element_type=jnp.float32) * (1.0 / 6.0)
        z7 = jnp.zeros((7, 256), jnp.float32)
        z15 = jnp.zeros((15, 256), jnp.float32)
        lower = jnp.concatenate(
            [wvc, z7, wfn, z7, w_ref[192:193, :], z15], axis=0)
        for h in range(2):
            wf1_ref[h] = w_ref[0:128, pl.ds(h * 128, 128)]
            wf2_ref[h] = lower[:, h * 128:(h + 1) * 128].astype(jnp.bfloat16)

    h = pl.program_id(1)
    for bi in range(_MB):
        hv = hv_ref[bi]
        a = a_ref[bi]
        y = (jnp.dot(hv, wf1_ref[h], preferred_element_type=jnp.float32)
             + jnp.dot(a, wf2_ref[h], preferred_element_type=jnp.float32)
             + b_ref[...])
        o_ref[:, bi, :] = jnp.where(y > 0.0, y,
                                    jnp.exp(jnp.minimum(y, 0.0)) - 1.0)


def _tc_call(hvar, ab, emb_vc, emb_func, w, b):
    return pl.pallas_call(
        _tc_body,
        grid=(_STEPS, 2),
        in_specs=[
            pl.BlockSpec((_MB, _L, 128), lambda i, h: (i, 0, 0)),
            pl.BlockSpec((_MB, _L, 128), lambda i, h: (i, 0, 0)),
            pl.BlockSpec((33, 32), lambda i, h: (0, 0)),
            pl.BlockSpec((65, 32), lambda i, h: (0, 0)),
            pl.BlockSpec((193, 256), lambda i, h: (0, 0)),
            pl.BlockSpec((1, 128), lambda i, h: (0, h)),
        ],
        out_specs=pl.BlockSpec((_L, _MB, 128), lambda i, h: (0, i, h)),
        out_shape=jax.ShapeDtypeStruct((_L, _B, _OUT), jnp.float32),
        scratch_shapes=[
            pltpu.VMEM((2, 128, 128), jnp.float32),
            pltpu.VMEM((2, 128, 128), jnp.bfloat16),
        ],
    )(hvar, ab, emb_vc, emb_func, w, b)


def kernel(x_omic, emb_var, emb_vc, emb_func, W, b):
    idx3 = _extract_idx(x_omic)
    ab = _extract_a(x_omic)
    idx = idx3.reshape(_B * _IDXB)
    hvar = _sc_gather(emb_var, idx)
    out = _tc_call(hvar, ab, emb_vc, emb_func, W, b.reshape(1, _OUT))
    return jnp.transpose(out, (1, 0, 2))


# speedup vs baseline: 1.8690x; 1.0076x over previous
import jax
import jax.numpy as jnp
from jax import lax
from jax.experimental import pallas as pl
from jax.experimental.pallas import tpu as pltpu
from jax.experimental.pallas import tpu_sc as plsc

_B, _L, _OUT = 128, 1425, 256
_N = _B * _L

_NC, _NS = 2, 16
_NW = _NC * _NS
_BPW = _B // _NW
_CHUNK = 128
_FULL = _L // _CHUNK
_TAIL = _L - _FULL * _CHUNK
_IDXB = 1440
_NBUF = 3
_NFK = _BPW * _FULL

_MB = 8
_STEPS = _B // _MB


def _sc_gather_body(table_hbm, idx_hbm, out_hbm, idx_v,
                    bufs, tbuf, sems, tsem):
    wid = lax.axis_index("s") * _NC + lax.axis_index("c")
    b0 = wid * _BPW

    pltpu.sync_copy(idx_hbm.at[pl.ds(wid * _BPW * _IDXB, _BPW * _IDXB)],
                    idx_v)

    def start_g(k, b):
        bl = k // _FULL
        c = k % _FULL
        iref = idx_v.at[pl.ds(bl * _IDXB + c * _CHUNK, _CHUNK)]
        pltpu.async_copy(table_hbm.at[iref], bufs[b], sems[b])

    def wait_g(b):
        iref = idx_v.at[pl.ds(0, _CHUNK)]
        pltpu.make_async_copy(table_hbm.at[iref], bufs[b], sems[b]).wait()

    for b in range(_NBUF):
        start_g(b, b)

    def body(k, carry):
        bl = k // _FULL
        c = k % _FULL
        for b in range(_NBUF):
            @pl.when(k % _NBUF == b)
            def _():
                wait_g(b)
                pltpu.sync_copy(
                    bufs[b], out_hbm.at[b0 + bl, pl.ds(c * _CHUNK, _CHUNK)])

                @pl.when(k + _NBUF < _NFK)
                def _():
                    start_g(k + _NBUF, b)
        return carry

    lax.fori_loop(0, _NFK, body, 0)

    for bl in range(_BPW):
        iref = idx_v.at[pl.ds(bl * _IDXB + _FULL * _CHUNK, _TAIL)]
        pltpu.async_copy(table_hbm.at[iref], tbuf, tsem).wait()
        pltpu.sync_copy(
            tbuf, out_hbm.at[b0 + bl, pl.ds(_FULL * _CHUNK, _TAIL)])


def _sc_gather(table, idx_flat):
    mesh = plsc.VectorSubcoreMesh(core_axis_name="c", subcore_axis_name="s")
    fn = pl.kernel(
        lambda table_hbm, idx_hbm, out_hbm, idx_v, u0, u1, u2, tb, s0, s1,
        s2, ts: _sc_gather_body(table_hbm, idx_hbm, out_hbm, idx_v,
                                (u0, u1, u2), tb, (s0, s1, s2), ts),
        out_type=jax.ShapeDtypeStruct((_B, _L, 128), jnp.float32),
        mesh=mesh,
        scratch_types=[
            pltpu.VMEM((_BPW * _IDXB,), jnp.int32),
            pltpu.VMEM((_CHUNK, 128), jnp.float32),
            pltpu.VMEM((_CHUNK, 128), jnp.float32),
            pltpu.VMEM((_CHUNK, 128), jnp.float32),
            pltpu.VMEM((_TAIL, 128), jnp.float32),
            pltpu.SemaphoreType.DMA,
            pltpu.SemaphoreType.DMA,
            pltpu.SemaphoreType.DMA,
            pltpu.SemaphoreType.DMA,
        ],
    )
    return fn(table, idx_flat)


def _extract_idx_body(x_ref, oi_ref):
    rows = [x_ref[bi, :, 0] for bi in range(8)]
    stacked = jnp.stack(rows, axis=0)
    padded = jnp.concatenate(
        [stacked, jnp.zeros((8, _IDXB - _L), jnp.float32)], axis=1)
    oi_ref[...] = padded.astype(jnp.int32)[None]


def _extract_idx(x_omic):
    return pl.pallas_call(
        _extract_idx_body,
        grid=(_B // 8,),
        in_specs=[pl.BlockSpec((8, _L, 9), lambda i: (i, 0, 0))],
        out_specs=pl.BlockSpec((1, 8, _IDXB), lambda i: (i, 0, 0)),
        out_shape=jax.ShapeDtypeStruct((_B // 8, 8, _IDXB), jnp.int32),
    )(x_omic)


def _extract_a_body(x_ref, oa_ref):
    iota = lax.broadcasted_iota(
        jnp.int32, (1, 128), 1).astype(jnp.bfloat16)
    one = jnp.bfloat16(1.0)
    zero = jnp.bfloat16(0.0)
    xb = [x_ref[0, :, k:k + 1].astype(jnp.bfloat16) for k in range(1, 9)]
    tgt = iota - jnp.bfloat16(40.0)
    t = [jnp.where(xb[0] == iota, one, zero)]
    t += [jnp.where(xb[1 + k] == tgt, one, zero) for k in range(6)]
    t += [xb[7] * jnp.where(iota == jnp.bfloat16(112.0), one, zero)
          + jnp.where(iota == jnp.bfloat16(113.0), one, zero)]
    oa_ref[0] = ((t[0] + t[1]) + (t[2] + t[3])) + ((t[4] + t[5])
                                                   + (t[6] + t[7]))


def _extract_a(x_omic):
    return pl.pallas_call(
        _extract_a_body,
        grid=(_B,),
        in_specs=[pl.BlockSpec((1, _L, 9), lambda i: (i, 0, 0))],
        out_specs=pl.BlockSpec((1, _L, 128), lambda i: (i, 0, 0)),
        out_shape=jax.ShapeDtypeStruct((_B, _L, 128), jnp.bfloat16),
    )(x_omic)


def _tc_body(hv_ref, a_ref, evc_ref, efn_ref, w_ref, b_ref, o_ref,
             wf1_ref, wf2_ref):
    @pl.when((pl.program_id(0) == 0) & (pl.program_id(1) == 0))
    def _():
        wvc = jnp.dot(evc_ref[...], w_ref[128:160, :],
                      preferred_element_type=jnp.float32)
        wfn = jnp.dot(efn_ref[...], w_ref[160:192, :],
                      preferred_element_type=jnp.float32) * (1.0 / 6.0)
        z7 = jnp.zeros((7, 256), jnp.float32)
        z14 = jnp.zeros((14, 256), jnp.float32)
        lower = jnp.concatenate(
            [wvc, z7, wfn, z7, w_ref[192:193, :], b_ref[...], z14],
            axis=0)
        for h in range(2):
            wf1_ref[h] = w_ref[0:128, pl.ds(h * 128, 128)]
            wf2_ref[h] = lower[:, h * 128:(h + 1) * 128].astype(jnp.bfloat16)

    h = pl.program_id(1)
    for bi in range(_MB):
        hv = hv_ref[bi]
        a = a_ref[bi]
        y = (jnp.dot(hv, wf1_ref[h], preferred_element_type=jnp.float32)
             + jnp.dot(a, wf2_ref[h], preferred_element_type=jnp.float32))
        o_ref[:, bi, :] = jnp.maximum(y, jnp.exp(jnp.minimum(y, 0.0)) - 1.0)


def _tc_call(hvar, ab, emb_vc, emb_func, w, b):
    return pl.pallas_call(
        _tc_body,
        grid=(_STEPS, 2),
        in_specs=[
            pl.BlockSpec((_MB, _L, 128), lambda i, h: (i, 0, 0)),
            pl.BlockSpec((_MB, _L, 128), lambda i, h: (i, 0, 0)),
            pl.BlockSpec((33, 32), lambda i, h: (0, 0)),
            pl.BlockSpec((65, 32), lambda i, h: (0, 0)),
            pl.BlockSpec((193, 256), lambda i, h: (0, 0)),
            pl.BlockSpec((1, 256), lambda i, h: (0, 0)),
        ],
        out_specs=pl.BlockSpec((_L, _MB, 128), lambda i, h: (0, i, h)),
        out_shape=jax.ShapeDtypeStruct((_L, _B, _OUT), jnp.float32),
        scratch_shapes=[
            pltpu.VMEM((2, 128, 128), jnp.float32),
            pltpu.VMEM((2, 128, 128), jnp.bfloat16),
        ],
    )(hvar, ab, emb_vc, emb_func, w, b)


def kernel(x_omic, emb_var, emb_vc, emb_func, W, b):
    idx3 = _extract_idx(x_omic)
    ab = _extract_a(x_omic)
    idx = idx3.reshape(_B * _IDXB)
    hvar = _sc_gather(emb_var, idx)
    out = _tc_call(hvar, ab, emb_vc, emb_func, W, b.reshape(1, _OUT))
    return jnp.transpose(out, (1, 0, 2))
